# baseline (speedup 1.0000x reference)
import functools

import jax
import jax.numpy as jnp
from jax.experimental import pallas as pl

_B = 8192
_BB = 32
_F32 = jnp.float32


def _row_body(db1_ref, dw2_ref, db2_ref, row_ref):
    r = jnp.maximum(db1_ref[...], 0.0)
    row_ref[...] = jax.lax.dot_general(
        r, dw2_ref[...], (((1,), (0,)), ((), ())),
        preferred_element_type=_F32) + db2_ref[...]


def _main_body(x_ref, cf_ref, w1_ref, b1_ref, w2_ref, b2_ref,
               cw1_ref, cb1_ref, cw2_ref, cb2_ref, row_ref,
               out_ref, fused_ref):
    bB = x_ref.shape[0]
    x = x_ref[...]
    zw = jnp.zeros((bB, 32, 1), _F32)
    xp = jnp.concatenate([zw, x, zw], axis=2)
    zh = jnp.zeros((bB, 1, 34), _F32)
    xp = jnp.concatenate([zh, xp, zh], axis=1)
    x3 = jnp.concatenate([xp[:, d:d + 32, :] for d in range(3)], axis=2)
    y1 = jax.lax.dot_general(
        x3.reshape(bB * 32, 102), w1_ref[...],
        (((1,), (0,)), ((), ())), preferred_element_type=_F32)
    y1 = jnp.maximum(y1 + b1_ref[...], 0.0)
    t = jnp.max(y1.reshape(bB * 16, 2, 1024), axis=1)
    t = jnp.max(t.reshape(bB * 16, 16, 2, 32), axis=2)
    h1 = t.reshape(bB, 16, 16, 32)
    zh2 = jnp.zeros((bB, 1, 16, 32), _F32)
    hp = jnp.concatenate([zh2, h1, zh2], axis=1)
    zc = jnp.zeros((bB, 16, 1, 32), _F32)
    blocks = []
    for d in range(3):
        r = hp[:, d:d + 16, :, :].reshape(bB, 16, 8, 2, 32)
        a0 = r[:, :, :, 0, :]
        a1 = r[:, :, :, 1, :]
        prev_a1 = jnp.concatenate([zc, a1[:, :, :-1, :]], axis=2)
        next_a0 = jnp.concatenate([a0[:, :, 1:, :], zc], axis=2)
        blocks += [prev_a1, a0, a1, next_a0]
    p2 = jnp.concatenate(blocks, axis=3)
    y2 = jax.lax.dot_general(
        p2.reshape(bB * 128, 384), w2_ref[...],
        (((1,), (0,)), ((), ())), preferred_element_type=_F32)
    y2 = jnp.maximum(y2 + b2_ref[...], 0.0)
    t2 = jnp.max(y2.reshape(bB, 8, 2, 8, 128), axis=2)
    t2 = jnp.max(t2.reshape(bB, 8, 8, 2, 64), axis=3)
    enc = jnp.mean(t2.reshape(bB, 64, 64), axis=1)
    cf = cf_ref[...]
    c = jax.lax.dot_general(cf, cw1_ref[...], (((1,), (0,)), ((), ())),
                            preferred_element_type=_F32)
    c = jnp.maximum(c + cb1_ref[...], 0.0)
    c = jax.lax.dot_general(c, cw2_ref[...], (((1,), (0,)), ((), ())),
                            preferred_element_type=_F32) + cb2_ref[...]
    fused_ref[...] = enc + c
    out_ref[...] = jnp.broadcast_to(row_ref[...], (bB, 4096))


def _build_w1(conv1_w):
    w1t = conv1_w[:, 0].transpose(1, 2, 0)
    wo = jnp.arange(32)
    dj = jnp.arange(3)
    w = jnp.zeros((3, 34, 32, 32), _F32)
    w = w.at[:, wo[None, :] + dj[:, None], wo[None, :], :].set(
        w1t[:, :, None, :])
    return w.reshape(102, 1024)


def _build_w2(conv2_w):
    w2t = conv2_w.transpose(2, 3, 1, 0)
    w = jnp.zeros((3, 4, 32, 2, 64), _F32)
    for q in range(2):
        for dj in range(3):
            w = w.at[:, q + dj, :, q, :].set(w2t[:, dj, :, :])
    return w.reshape(384, 128)


@functools.partial(jax.jit, static_argnames=())
def kernel(state, conceptual_features, conv1_w, conv1_b, conv2_w, conv2_b,
           ce_w1, ce_b1, ce_w2, ce_b2, router_w, dec_w1, dec_b1, dec_w2,
           dec_b2):
    del router_w, dec_w1
    x = state.reshape(_B, 32, 32)
    w1 = _build_w1(conv1_w)
    b1t = jnp.tile(conv1_b, 32).reshape(1, 1024)
    w2 = _build_w2(conv2_w)
    b2t = jnp.tile(conv2_b, 2).reshape(1, 128)

    row = pl.pallas_call(
        _row_body,
        out_shape=jax.ShapeDtypeStruct((1, 4096), _F32),
    )(dec_b1.reshape(1, 256), dec_w2, dec_b2.reshape(1, 4096))

    grid = (_B // _BB,)
    zero = lambda i: (0, 0)
    out, fused = pl.pallas_call(
        _main_body,
        grid=grid,
        in_specs=[
            pl.BlockSpec((_BB, 32, 32), lambda i: (i, 0, 0)),
            pl.BlockSpec((_BB, 64), lambda i: (i, 0)),
            pl.BlockSpec((102, 1024), zero),
            pl.BlockSpec((1, 1024), zero),
            pl.BlockSpec((384, 128), zero),
            pl.BlockSpec((1, 128), zero),
            pl.BlockSpec((64, 64), zero),
            pl.BlockSpec((1, 64), zero),
            pl.BlockSpec((64, 64), zero),
            pl.BlockSpec((1, 64), zero),
            pl.BlockSpec((1, 4096), zero),
        ],
        out_specs=[
            pl.BlockSpec((_BB, 4096), lambda i: (i, 0)),
            pl.BlockSpec((_BB, 64), lambda i: (i, 0)),
        ],
        out_shape=[
            jax.ShapeDtypeStruct((_B, 4096), _F32),
            jax.ShapeDtypeStruct((_B, 64), _F32),
        ],
    )(x, conceptual_features, w1, b1t, w2, b2t,
      ce_w1, ce_b1.reshape(1, 64), ce_w2, ce_b2.reshape(1, 64), row)

    return (out, fused, jnp.zeros((), _F32))


# speedup vs baseline: 1.4124x; 1.4124x over previous
import jax
import jax.numpy as jnp
import numpy as np
from jax.experimental import pallas as pl
from jax.experimental.pallas import tpu as pltpu

_B = 8192
_BB = 32
_F32 = jnp.float32


def _dot(a, b):
    return jax.lax.dot_general(a, b, (((1,), (0,)), ((), ())),
                               preferred_element_type=_F32)


def _dot00(a, b):
    return jax.lax.dot_general(a, b, (((0,), (0,)), ((), ())),
                               preferred_element_type=_F32)


def _row_body(db1_ref, dw2_ref, db2_ref, row_ref):
    row_ref[...] = _dot(jnp.maximum(db1_ref[...], 0.0),
                        dw2_ref[...]) + db2_ref[...]


def _shift_w(x, delta, bb):
    n = x.shape[1]
    if delta == 0:
        return x
    z = jnp.zeros((x.shape[0], abs(delta) * bb), _F32)
    if delta > 0:
        return jnp.concatenate([x[:, delta * bb:], z], axis=1)
    return jnp.concatenate([z, x[:, :n + delta * bb]], axis=1)


def _main_body(x_ref, cf_ref, w1_ref, b1_ref, w2_ref, b2_ref,
               cw1_ref, cb1_ref, cw2_ref, cb2_ref, row_ref,
               out_ref, fused_ref):
    bb = _BB
    n1 = 32 * bb
    x = x_ref[...].reshape(32, n1)
    xs = jnp.concatenate([_shift_w(x, d - 1, bb) for d in range(3)], axis=0)
    y1 = _dot(w1_ref[...], xs)
    y1 = jnp.maximum(y1 + b1_ref[...], 0.0)
    t = jnp.max(y1.reshape(16, 2, 32, n1), axis=1).reshape(512, n1)
    n2 = 16 * bb
    t = jnp.max(t.reshape(512, 16, 2, bb), axis=2).reshape(512, n2)
    zr = jnp.zeros((32, n2), _F32)
    hp = jnp.concatenate([zr, t, zr], axis=0)
    sh = [_shift_w(hp, d - 1, bb) for d in range(3)]
    w2all = w2_ref[...]
    groups = []
    for g in range(8):
        acc = _dot00(w2all[0], sh[0][64 * g:64 * g + 128])
        acc += _dot00(w2all[1], sh[1][64 * g:64 * g + 128])
        acc += _dot00(w2all[2], sh[2][64 * g:64 * g + 128])
        groups.append(acc)
    y2 = jnp.concatenate(groups, axis=0)
    y2 = jnp.maximum(y2 + b2_ref[...], 0.0)
    t2 = jnp.max(y2.reshape(8, 2, 64, n2), axis=1).reshape(512, n2)
    n3 = 8 * bb
    t2 = jnp.max(t2.reshape(512, 8, 2, bb), axis=2).reshape(512, n3)
    s = jnp.sum(t2.reshape(8, 64, n3), axis=0)
    enc = jnp.sum(s.reshape(64, 8, bb), axis=1) * (1.0 / 64.0)
    cf = cf_ref[...].reshape(64, bb)
    c = jnp.maximum(_dot(cw1_ref[...], cf) + cb1_ref[...], 0.0)
    c = _dot(cw2_ref[...], c) + cb2_ref[...]
    fused_ref[...] = (enc + c).reshape(1, 64, bb)
    out_ref[...] = jnp.broadcast_to(row_ref[...], (bb, 4096))


def _build_w1(conv1_w):
    bm = jnp.zeros((32, 32, 3, 32), _F32)
    h = np.arange(32)
    for di in range(3):
        hs = h[(h + di - 1 >= 0) & (h + di - 1 <= 31)]
        val = conv1_w[:, 0, di, :]
        bm = bm.at[hs, :, :, hs + di - 1].set(val[None])
    return bm.reshape(1024, 96)


def _build_w2(conv2_w):
    wm = jnp.zeros((3, 4, 32, 2, 64), _F32)
    for q in range(2):
        for hh in range(4):
            di = hh - q
            if 0 <= di <= 2:
                wm = wm.at[:, hh, :, q, :].set(
                    conv2_w[:, :, di, :].transpose(2, 1, 0))
    return wm.reshape(3, 128, 128)


def kernel(state, conceptual_features, conv1_w, conv1_b, conv2_w, conv2_b,
           ce_w1, ce_b1, ce_w2, ce_b2, router_w, dec_w1, dec_b1, dec_w2,
           dec_b2):
    del router_w, dec_w1
    nblk = _B // _BB
    xt = state.reshape(nblk, _BB, 32, 32).transpose(0, 2, 3, 1)
    xt = xt.reshape(nblk, 32, 32 * _BB)
    cft = conceptual_features.reshape(nblk, _BB, 64).transpose(0, 2, 1)
    w1 = _build_w1(conv1_w)
    b1c = jnp.tile(conv1_b, 32).reshape(1024, 1)
    w2 = _build_w2(conv2_w)
    b2c = jnp.tile(conv2_b, 16).reshape(1024, 1)

    row = pl.pallas_call(
        _row_body,
        out_shape=jax.ShapeDtypeStruct((1, 4096), _F32),
    )(dec_b1.reshape(1, 256), dec_w2, dec_b2.reshape(1, 4096))

    zero2 = lambda i: (0, 0)
    out, fused_t = pl.pallas_call(
        _main_body,
        grid=(nblk,),
        in_specs=[
            pl.BlockSpec((1, 32, 32 * _BB), lambda i: (i, 0, 0)),
            pl.BlockSpec((1, 64, _BB), lambda i: (i, 0, 0)),
            pl.BlockSpec((1024, 96), zero2),
            pl.BlockSpec((1024, 1), zero2),
            pl.BlockSpec((3, 128, 128), lambda i: (0, 0, 0)),
            pl.BlockSpec((1024, 1), zero2),
            pl.BlockSpec((64, 64), zero2),
            pl.BlockSpec((64, 1), zero2),
            pl.BlockSpec((64, 64), zero2),
            pl.BlockSpec((64, 1), zero2),
            pl.BlockSpec((1, 4096), zero2),
        ],
        out_specs=[
            pl.BlockSpec((_BB, 4096), lambda i: (i, 0)),
            pl.BlockSpec((1, 64, _BB), lambda i: (i, 0, 0)),
        ],
        out_shape=[
            jax.ShapeDtypeStruct((_B, 4096), _F32),
            jax.ShapeDtypeStruct((nblk, 64, _BB), _F32),
        ],
        compiler_params=pltpu.CompilerParams(
            dimension_semantics=("arbitrary",)),
    )(xt, cft, w1, b1c, w2, b2c,
      ce_w1.T, ce_b1.reshape(64, 1), ce_w2.T, ce_b2.reshape(64, 1), row)

    fused = fused_t.transpose(0, 2, 1).reshape(_B, 64)
    return (out, fused, jnp.zeros((), _F32))


# speedup vs baseline: 2.6564x; 1.8807x over previous
import jax
import jax.numpy as jnp
import numpy as np
from jax.experimental import pallas as pl
from jax.experimental.pallas import tpu as pltpu

_B = 8192
_BB = 64
_F32 = jnp.float32


def _dot(a, b):
    return jax.lax.dot_general(a, b, (((1,), (0,)), ((), ())),
                               preferred_element_type=_F32)


def _dot00(a, b):
    return jax.lax.dot_general(a, b, (((0,), (0,)), ((), ())),
                               preferred_element_type=_F32)


def _row_body(db1_ref, dw2_ref, db2_ref, row_ref):
    row_ref[...] = _dot(jnp.maximum(db1_ref[...], 0.0),
                        dw2_ref[...]) + db2_ref[...]


def _shift_w(x, delta, bb):
    n = x.shape[1]
    if delta == 0:
        return x
    z = jnp.zeros((x.shape[0], abs(delta) * bb), _F32)
    if delta > 0:
        return jnp.concatenate([x[:, delta * bb:], z], axis=1)
    return jnp.concatenate([z, x[:, :n + delta * bb]], axis=1)


def _main_body(x_ref, cf_ref, w1_ref, b1_ref, w2_ref, b2_ref,
               cw1_ref, cb1_ref, cw2_ref, cb2_ref, row_ref,
               out_ref, fused_ref):
    bb = _BB
    n1 = 32 * bb
    n2 = 16 * bb
    x = x_ref[...].reshape(32, n1)
    xs = jnp.concatenate([_shift_w(x, d - 1, bb) for d in range(3)], axis=0)
    halves = []
    for hf in range(2):
        y1 = _dot(w1_ref[...], xs[:, hf * n2:(hf + 1) * n2])
        t = jnp.max(y1.reshape(16, 2, 32, n2), axis=1).reshape(512, n2)
        t = jnp.max(t.reshape(512, 8, 2, bb), axis=2).reshape(512, n2 // 2)
        halves.append(t)
    t = jnp.concatenate(halves, axis=1)
    t = jnp.maximum(t + b1_ref[...], 0.0)
    zr = jnp.zeros((32, n2), _F32)
    hp = jnp.concatenate([zr, t, zr], axis=0)
    sh = [_shift_w(hp, d - 1, bb) for d in range(3)]
    w2all = w2_ref[...]
    b2c = b2_ref[...]
    enc = jnp.zeros((64, bb), _F32)
    for g in range(8):
        acc = _dot00(w2all[0], sh[0][64 * g:64 * g + 128])
        acc += _dot00(w2all[1], sh[1][64 * g:64 * g + 128])
        acc += _dot00(w2all[2], sh[2][64 * g:64 * g + 128])
        p = jnp.max(acc.reshape(2, 64, n2), axis=0)
        p = jnp.max(p.reshape(64, 8, 2, bb), axis=2)
        p = jnp.maximum(p + b2c.reshape(64, 1, 1), 0.0)
        enc = enc + jnp.sum(p, axis=1)
    enc = enc * (1.0 / 64.0)
    cf = cf_ref[...].reshape(64, bb)
    c = jnp.maximum(_dot(cw1_ref[...], cf) + cb1_ref[...], 0.0)
    c = _dot(cw2_ref[...], c) + cb2_ref[...]
    fused_ref[...] = (enc + c).reshape(1, 64, bb)
    out_ref[...] = jnp.broadcast_to(row_ref[...], (bb, 4096))


def _build_w1(conv1_w):
    bm = jnp.zeros((32, 32, 3, 32), _F32)
    h = np.arange(32)
    for di in range(3):
        hs = h[(h + di - 1 >= 0) & (h + di - 1 <= 31)]
        val = conv1_w[:, 0, di, :]
        bm = bm.at[hs, :, :, hs + di - 1].set(val[None])
    return bm.reshape(1024, 96)


def _build_w2(conv2_w):
    wm = jnp.zeros((3, 4, 32, 2, 64), _F32)
    for q in range(2):
        for hh in range(4):
            di = hh - q
            if 0 <= di <= 2:
                wm = wm.at[:, hh, :, q, :].set(
                    conv2_w[:, :, di, :].transpose(2, 1, 0))
    return wm.reshape(3, 128, 128)


def kernel(state, conceptual_features, conv1_w, conv1_b, conv2_w, conv2_b,
           ce_w1, ce_b1, ce_w2, ce_b2, router_w, dec_w1, dec_b1, dec_w2,
           dec_b2):
    del router_w, dec_w1
    nblk = _B // _BB
    xt = state.reshape(nblk, _BB, 32, 32).transpose(0, 2, 3, 1)
    xt = xt.reshape(nblk, 32, 32 * _BB)
    cft = conceptual_features.reshape(nblk, _BB, 64).transpose(0, 2, 1)
    w1 = _build_w1(conv1_w)
    b1c = jnp.tile(conv1_b, 16).reshape(512, 1)
    w2 = _build_w2(conv2_w)

    row = pl.pallas_call(
        _row_body,
        out_shape=jax.ShapeDtypeStruct((1, 4096), _F32),
    )(dec_b1.reshape(1, 256), dec_w2, dec_b2.reshape(1, 4096))

    zero2 = lambda i: (0, 0)
    out, fused_t = pl.pallas_call(
        _main_body,
        grid=(nblk,),
        in_specs=[
            pl.BlockSpec((1, 32, 32 * _BB), lambda i: (i, 0, 0)),
            pl.BlockSpec((1, 64, _BB), lambda i: (i, 0, 0)),
            pl.BlockSpec((1024, 96), zero2),
            pl.BlockSpec((512, 1), zero2),
            pl.BlockSpec((3, 128, 128), lambda i: (0, 0, 0)),
            pl.BlockSpec((64, 1), zero2),
            pl.BlockSpec((64, 64), zero2),
            pl.BlockSpec((64, 1), zero2),
            pl.BlockSpec((64, 64), zero2),
            pl.BlockSpec((64, 1), zero2),
            pl.BlockSpec((1, 4096), zero2),
        ],
        out_specs=[
            pl.BlockSpec((_BB, 4096), lambda i: (i, 0)),
            pl.BlockSpec((1, 64, _BB), lambda i: (i, 0, 0)),
        ],
        out_shape=[
            jax.ShapeDtypeStruct((_B, 4096), _F32),
            jax.ShapeDtypeStruct((nblk, 64, _BB), _F32),
        ],
        compiler_params=pltpu.CompilerParams(
            dimension_semantics=("arbitrary",)),
    )(xt, cft, w1, b1c, w2, conv2_b.reshape(64, 1),
      ce_w1.T, ce_b1.reshape(64, 1), ce_w2.T, ce_b2.reshape(64, 1), row)

    fused = fused_t.transpose(0, 2, 1).reshape(_B, 64)
    return (out, fused, jnp.zeros((), _F32))
